# Initial kernel scaffold; baseline (speedup 1.0000x reference)
import jax
import jax.numpy as jnp
from jax.experimental import pallas as pl


def kernel(text_embedding, W1, b1, W2, b2):
    raise NotImplementedError("write your pallas kernel here")


# speedup vs baseline: 1.6941x; 1.6941x over previous
import functools

import jax
import jax.numpy as jnp
from jax.experimental import pallas as pl

EMBED = 4096
L = 32
E = 64
K = 8
TM = 256


def _router_kernel(x_ref, w1_ref, b1_ref, w2_ref, b2_ref, logits_ref, mask_ref):
    h = jnp.maximum(
        jnp.dot(x_ref[...], w1_ref[...], preferred_element_type=jnp.float32)
        + b1_ref[...],
        0.0,
    )
    w2 = w2_ref[...]
    b2 = b2_ref[...]
    for g in range(L):
        lg = jnp.dot(h, w2[:, g, :], preferred_element_type=jnp.float32) + b2[g]
        logits_ref[:, g, :] = lg
    logits3 = logits_ref[...]
    work = logits3
    for _ in range(K):
        m = jnp.max(work, axis=-1, keepdims=True)
        work = jnp.where(work >= m, -jnp.inf, work)
    mask_ref[...] = (logits3 >= m).astype(jnp.float32)


def kernel(text_embedding, W1, b1, W2, b2):
    B = text_embedding.shape[0]
    hid = W1.shape[1]
    w2r = W2.reshape(hid, L, E)
    b2r = b2.reshape(L, E)
    b1r = b1.reshape(1, hid)
    grid = (B // TM,)
    logits, mask = pl.pallas_call(
        _router_kernel,
        grid=grid,
        in_specs=[
            pl.BlockSpec((TM, EMBED), lambda i: (i, 0)),
            pl.BlockSpec((EMBED, hid), lambda i: (0, 0)),
            pl.BlockSpec((1, hid), lambda i: (0, 0)),
            pl.BlockSpec((hid, L, E), lambda i: (0, 0, 0)),
            pl.BlockSpec((L, E), lambda i: (0, 0)),
        ],
        out_specs=[
            pl.BlockSpec((TM, L, E), lambda i: (i, 0, 0)),
            pl.BlockSpec((TM, L, E), lambda i: (i, 0, 0)),
        ],
        out_shape=[
            jax.ShapeDtypeStruct((B, L, E), jnp.float32),
            jax.ShapeDtypeStruct((B, L, E), jnp.float32),
        ],
    )(text_embedding, W1, b1r, w2r, b2r)
    return (mask, logits)
